# baseline (speedup 1.0000x reference)
import functools

import jax
import jax.numpy as jnp
from jax import lax
from jax.experimental import pallas as pl
from jax.experimental.pallas import tpu as pltpu
from jax.experimental.pallas import tpu_sc as plsc

VOCAB = 1000000
D = 32
B = 4096
H = 200
N = B * H

NC = 2
NS = 16
NW = NC * NS

NCHUNK = 4
CN = N // NCHUNK
PER_W = CN // NW
G = 128
NG = PER_W // G
K = 10
GROUP = K * G
NGROUP = PER_W // GROUP
assert NGROUP * GROUP == PER_W and NG * G == PER_W


def _emb_body(x_hbm, tab_hbm, out_hbm, idx_v, rows_v, sem):
    c = lax.axis_index("c")
    s = lax.axis_index("s")
    wid = s * NC + c
    pltpu.sync_copy(x_hbm.at[pl.ds(wid * NG, NG)], idx_v)
    out_base = wid * PER_W

    def group(g, carry):
        copies = []
        for k in range(K):
            cp = pltpu.async_copy(
                tab_hbm.at[idx_v.at[g * K + k]],
                rows_v.at[pl.ds(k * G, G)],
                sem,
            )
            copies.append(cp)
        for cp in copies:
            cp.wait()
        pltpu.sync_copy(rows_v, out_hbm.at[pl.ds(out_base + g * GROUP, GROUP)])
        return carry

    lax.fori_loop(0, NGROUP, group, 0)


@functools.partial(
    pl.kernel,
    mesh=plsc.VectorSubcoreMesh(core_axis_name="c", subcore_axis_name="s"),
    out_type=jax.ShapeDtypeStruct((CN, D), jnp.float32),
    scratch_types=[
        pltpu.VMEM((NG, G), jnp.int32),
        pltpu.VMEM((GROUP, D), jnp.float32),
        pltpu.SemaphoreType.DMA,
    ],
    compiler_params=pltpu.CompilerParams(use_tc_tiling_on_sc=False),
)
def _emb(x_hbm, tab_hbm, out_hbm, idx_v, rows_v, sem):
    _emb_body(x_hbm, tab_hbm, out_hbm, idx_v, rows_v, sem)


def kernel(X, table):
    xt = X.astype(jnp.int32).T
    hc = H // NCHUNK
    parts = []
    for i in range(NCHUNK):
        xi = lax.slice_in_dim(xt, i * hc, (i + 1) * hc)
        xi = xi.reshape(CN // G, G)
        oi = _emb(xi, table)
        parts.append(jnp.transpose(oi.reshape(hc, B, D), (1, 0, 2)))
    return jnp.concatenate(parts, axis=1)
